# baseline (speedup 1.0000x reference)
import functools

import jax
import jax.numpy as jnp
from jax import lax
from jax.experimental import pallas as pl
from jax.experimental.pallas import tpu as pltpu
from jax.experimental.pallas import tpu_sc as plsc

N = 10000
E = 320000
D = 128

NC = 2
NS = 16
NW = NC * NS
CHUNK = 50
CHUNKS = 200
E_TILE = CHUNK * CHUNKS
ROWS_SUB = 640
N_PAD = NS * ROWS_SUB
NBUF = 5
LOOK = 3
H = 5
CH = CHUNKS // H
DEG_RING = 4
assert CH % NBUF == 0 and 1 <= LOOK < NBUF

_MESH = plsc.VectorSubcoreMesh(core_axis_name="core", subcore_axis_name="subcore")


@functools.partial(
    pl.kernel,
    out_type=jax.ShapeDtypeStruct((NC, N_PAD, D), jnp.float32),
    mesh=_MESH,
    scratch_types=[
        pltpu.VMEM((CHUNKS, CHUNK), jnp.int32),
        pltpu.VMEM((CHUNK, D), jnp.float32),
        pltpu.VMEM_SHARED((N_PAD, D), jnp.float32),
    ] + [pltpu.SemaphoreType.DMA] * DEG_RING,
)
def _sc_degree(edges_hbm, ones_hbm, zeros_hbm, out_hbm, dst_v, ones_v, acc,
               *sems):
    c = lax.axis_index("core")
    s = lax.axis_index("subcore")
    wid = c * NS + s
    pltpu.sync_copy(zeros_hbm, acc.at[pl.ds(s * ROWS_SUB, ROWS_SUB)])
    pltpu.sync_copy(ones_hbm, ones_v)
    pltpu.sync_copy(edges_hbm.at[1, wid], dst_v)
    plsc.subcore_barrier()

    @pl.loop(0, CHUNKS, step=DEG_RING)
    def _(jo):
        for b in range(DEG_RING):
            j = jo + b

            @pl.when(jo > 0)
            def _():
                pltpu.make_async_copy(ones_v, acc.at[dst_v.at[j]],
                                      sems[b]).wait()

            pltpu.async_copy(ones_v, acc.at[dst_v.at[j]], sems[b], add=True)

    for b in range(DEG_RING):
        pltpu.make_async_copy(ones_v, acc.at[dst_v.at[b]], sems[b]).wait()

    plsc.subcore_barrier()
    pltpu.sync_copy(acc.at[pl.ds(s * ROWS_SUB, ROWS_SUB)],
                    out_hbm.at[c, pl.ds(s * ROWS_SUB, ROWS_SUB)])


@functools.partial(
    pl.kernel,
    out_type=jax.ShapeDtypeStruct((NC, N_PAD, D), jnp.float32),
    mesh=_MESH,
    scratch_types=[
        pltpu.VMEM((CH, CHUNK), jnp.int32),
        pltpu.VMEM((CH, CHUNK), jnp.int32),
        pltpu.VMEM((NBUF, CHUNK, D), jnp.float32),
        pltpu.VMEM_SHARED((N_PAD, D), jnp.float32),
    ] + [pltpu.SemaphoreType.DMA] * (2 * NBUF),
)
def _sc_aggregate(g_hbm, edges_hbm, zeros_hbm, out_hbm,
                  src_v, dst_v, rows_v, acc, *sems):
    gsem = sems[:NBUF]
    ssem = sems[NBUF:]
    c = lax.axis_index("core")
    s = lax.axis_index("subcore")
    wid = c * NS + s
    pltpu.sync_copy(zeros_hbm, acc.at[pl.ds(s * ROWS_SUB, ROWS_SUB)])
    plsc.subcore_barrier()

    for h in range(H):
        pltpu.sync_copy(edges_hbm.at[0, wid, pl.ds(h * CH, CH)], src_v)
        pltpu.sync_copy(edges_hbm.at[1, wid, pl.ds(h * CH, CH)], dst_v)
        for b in range(LOOK):
            pltpu.async_copy(g_hbm.at[src_v.at[b]], rows_v.at[b], gsem[b])

        @pl.loop(0, CH, step=NBUF)
        def _(jo):
            for b in range(NBUF):
                j = jo + b
                bl = (b + LOOK) % NBUF

                @pl.when(j >= NBUF - LOOK)
                def _():
                    pltpu.make_async_copy(rows_v.at[bl], acc.at[dst_v.at[j]],
                                          ssem[bl]).wait()

                jg = jnp.where(j + LOOK >= CH, 0, j + LOOK)
                pltpu.async_copy(g_hbm.at[src_v.at[jg]], rows_v.at[bl],
                                 gsem[bl])
                pltpu.make_async_copy(g_hbm.at[src_v.at[j]], rows_v.at[b],
                                      gsem[b]).wait()
                pltpu.async_copy(rows_v.at[b], acc.at[dst_v.at[j]], ssem[b],
                                 add=True)

        for t in range(CH + LOOK - NBUF, CH):
            pltpu.make_async_copy(rows_v.at[t % NBUF], acc.at[dst_v.at[0]],
                                  ssem[t % NBUF]).wait()
        for t in range(CH, CH + LOOK):
            pltpu.make_async_copy(g_hbm.at[src_v.at[0]], rows_v.at[t % NBUF],
                                  gsem[t % NBUF]).wait()

    plsc.subcore_barrier()
    pltpu.sync_copy(acc.at[pl.ds(s * ROWS_SUB, ROWS_SUB)],
                    out_hbm.at[c, pl.ds(s * ROWS_SUB, ROWS_SUB)])


def _dinv_from_counts(cnt_ref):
    s = (cnt_ref[0] + cnt_ref[1])[:, 0:1].astype(jnp.float32)
    return lax.rsqrt(1.0 + s)


def _tc_first(cnt_ref, x_ref, w_ref, g_ref):
    dinv = _dinv_from_counts(cnt_ref)
    h = jnp.dot(x_ref[...], w_ref[...], preferred_element_type=jnp.float32)
    g_ref[pl.ds(0, N)] = dinv[:N] * h
    g_ref[pl.ds(N, N_PAD - N)] = jnp.zeros((N_PAD - N, D), jnp.float32)


def _tc_mid(cnt_ref, a_ref, g_ref, b_ref, w_ref, g2_ref):
    dinv = _dinv_from_counts(cnt_ref)
    z = dinv * (a_ref[0] + a_ref[1] + g_ref[...]) + b_ref[...]
    z = jnp.maximum(z, 0.0)
    h = jnp.dot(z, w_ref[...], preferred_element_type=jnp.float32)
    g2_ref[...] = dinv * h


def _tc_last(cnt_ref, a_ref, g_ref, b_ref, out_ref):
    dinv = _dinv_from_counts(cnt_ref)
    out_ref[...] = dinv * (a_ref[0] + a_ref[1] + g_ref[...]) + b_ref[...]


def _call_tc(body, *args):
    return pl.pallas_call(
        body,
        out_shape=jax.ShapeDtypeStruct((N_PAD, D), jnp.float32),
    )(*args)


def kernel(x, edge_index, W1, b1, W2, b2):
    edges = edge_index.astype(jnp.int32).reshape(2, NW, CHUNKS, CHUNK)

    onesD = jnp.ones((CHUNK, D), jnp.float32)
    zerosD = jnp.zeros((ROWS_SUB, D), jnp.float32)
    b1r = b1.reshape(1, D)
    b2r = b2.reshape(1, D)

    cnt = _sc_degree(edges, onesD, zerosD)
    g1 = _call_tc(_tc_first, cnt, x, W1)
    a1 = _sc_aggregate(g1, edges, zerosD)
    g2 = _call_tc(_tc_mid, cnt, a1, g1, b1r, W2)
    a2 = _sc_aggregate(g2, edges, zerosD)
    out = _call_tc(_tc_last, cnt, a2, g2, b2r)
    return out[:N]


# speedup vs baseline: 1.0140x; 1.0140x over previous
import functools

import jax
import jax.numpy as jnp
from jax import lax
from jax.experimental import pallas as pl
from jax.experimental.pallas import tpu as pltpu
from jax.experimental.pallas import tpu_sc as plsc

N = 10000
E = 320000
D = 128

NC = 2
NS = 16
NW = NC * NS
CHUNK = 125
CHUNKS = 80
E_TILE = CHUNK * CHUNKS
ROWS_SUB = 640
N_PAD = NS * ROWS_SUB
NBUF = 2
LOOK = 1
H = 2
CH = CHUNKS // H
DEG_RING = 4
assert CH % NBUF == 0 and 1 <= LOOK < NBUF

_MESH = plsc.VectorSubcoreMesh(core_axis_name="core", subcore_axis_name="subcore")


@functools.partial(
    pl.kernel,
    out_type=jax.ShapeDtypeStruct((NC, N_PAD, D), jnp.float32),
    mesh=_MESH,
    scratch_types=[
        pltpu.VMEM((CHUNKS, CHUNK), jnp.int32),
        pltpu.VMEM((CHUNK, D), jnp.float32),
        pltpu.VMEM_SHARED((N_PAD, D), jnp.float32),
    ] + [pltpu.SemaphoreType.DMA] * DEG_RING,
)
def _sc_degree(edges_hbm, ones_hbm, zeros_hbm, out_hbm, dst_v, ones_v, acc,
               *sems):
    c = lax.axis_index("core")
    s = lax.axis_index("subcore")
    wid = c * NS + s
    pltpu.sync_copy(zeros_hbm, acc.at[pl.ds(s * ROWS_SUB, ROWS_SUB)])
    pltpu.sync_copy(ones_hbm, ones_v)
    pltpu.sync_copy(edges_hbm.at[1, wid], dst_v)
    plsc.subcore_barrier()

    @pl.loop(0, CHUNKS, step=DEG_RING)
    def _(jo):
        for b in range(DEG_RING):
            j = jo + b

            @pl.when(jo > 0)
            def _():
                pltpu.make_async_copy(ones_v, acc.at[dst_v.at[j]],
                                      sems[b]).wait()

            pltpu.async_copy(ones_v, acc.at[dst_v.at[j]], sems[b], add=True)

    for b in range(DEG_RING):
        pltpu.make_async_copy(ones_v, acc.at[dst_v.at[b]], sems[b]).wait()

    plsc.subcore_barrier()
    pltpu.sync_copy(acc.at[pl.ds(s * ROWS_SUB, ROWS_SUB)],
                    out_hbm.at[c, pl.ds(s * ROWS_SUB, ROWS_SUB)])


@functools.partial(
    pl.kernel,
    out_type=jax.ShapeDtypeStruct((NC, N_PAD, D), jnp.float32),
    mesh=_MESH,
    scratch_types=[
        pltpu.VMEM((CH, CHUNK), jnp.int32),
        pltpu.VMEM((CH, CHUNK), jnp.int32),
        pltpu.VMEM((NBUF, CHUNK, D), jnp.float32),
        pltpu.VMEM_SHARED((N_PAD, D), jnp.float32),
    ] + [pltpu.SemaphoreType.DMA] * (2 * NBUF),
)
def _sc_aggregate(g_hbm, edges_hbm, zeros_hbm, out_hbm,
                  src_v, dst_v, rows_v, acc, *sems):
    gsem = sems[:NBUF]
    ssem = sems[NBUF:]
    c = lax.axis_index("core")
    s = lax.axis_index("subcore")
    wid = c * NS + s
    pltpu.sync_copy(zeros_hbm, acc.at[pl.ds(s * ROWS_SUB, ROWS_SUB)])
    plsc.subcore_barrier()

    for h in range(H):
        pltpu.sync_copy(edges_hbm.at[0, wid, pl.ds(h * CH, CH)], src_v)
        pltpu.sync_copy(edges_hbm.at[1, wid, pl.ds(h * CH, CH)], dst_v)
        for b in range(LOOK):
            pltpu.async_copy(g_hbm.at[src_v.at[b]], rows_v.at[b], gsem[b])

        @pl.loop(0, CH, step=NBUF)
        def _(jo):
            for b in range(NBUF):
                j = jo + b
                bl = (b + LOOK) % NBUF

                @pl.when(j >= NBUF - LOOK)
                def _():
                    pltpu.make_async_copy(rows_v.at[bl], acc.at[dst_v.at[j]],
                                          ssem[bl]).wait()

                jg = jnp.where(j + LOOK >= CH, 0, j + LOOK)
                pltpu.async_copy(g_hbm.at[src_v.at[jg]], rows_v.at[bl],
                                 gsem[bl])
                pltpu.make_async_copy(g_hbm.at[src_v.at[j]], rows_v.at[b],
                                      gsem[b]).wait()
                pltpu.async_copy(rows_v.at[b], acc.at[dst_v.at[j]], ssem[b],
                                 add=True)

        for t in range(CH + LOOK - NBUF, CH):
            pltpu.make_async_copy(rows_v.at[t % NBUF], acc.at[dst_v.at[0]],
                                  ssem[t % NBUF]).wait()
        for t in range(CH, CH + LOOK):
            pltpu.make_async_copy(g_hbm.at[src_v.at[0]], rows_v.at[t % NBUF],
                                  gsem[t % NBUF]).wait()

    plsc.subcore_barrier()
    pltpu.sync_copy(acc.at[pl.ds(s * ROWS_SUB, ROWS_SUB)],
                    out_hbm.at[c, pl.ds(s * ROWS_SUB, ROWS_SUB)])


def _dinv_from_counts(cnt_ref):
    s = (cnt_ref[0] + cnt_ref[1])[:, 0:1].astype(jnp.float32)
    return lax.rsqrt(1.0 + s)


def _tc_matmul(x_ref, w_ref, h_ref):
    h_ref[...] = jnp.dot(x_ref[...], w_ref[...],
                         preferred_element_type=jnp.float32)


def _tc_scale(cnt_ref, h_ref, g_ref):
    dinv = _dinv_from_counts(cnt_ref)
    g_ref[pl.ds(0, N)] = dinv[:N] * h_ref[...]
    g_ref[pl.ds(N, N_PAD - N)] = jnp.zeros((N_PAD - N, D), jnp.float32)


def _tc_mid(cnt_ref, a_ref, g_ref, b_ref, w_ref, g2_ref):
    dinv = _dinv_from_counts(cnt_ref)
    z = dinv * (a_ref[0] + a_ref[1] + g_ref[...]) + b_ref[...]
    z = jnp.maximum(z, 0.0)
    h = jnp.dot(z, w_ref[...], preferred_element_type=jnp.float32)
    g2_ref[...] = dinv * h


def _tc_last(cnt_ref, a_ref, g_ref, b_ref, out_ref):
    dinv = _dinv_from_counts(cnt_ref)
    out_ref[...] = dinv * (a_ref[0] + a_ref[1] + g_ref[...]) + b_ref[...]


def _call_tc(body, *args):
    return pl.pallas_call(
        body,
        out_shape=jax.ShapeDtypeStruct((N_PAD, D), jnp.float32),
    )(*args)


def kernel(x, edge_index, W1, b1, W2, b2):
    edges = edge_index.astype(jnp.int32).reshape(2, NW, CHUNKS, CHUNK)

    onesD = jnp.ones((CHUNK, D), jnp.float32)
    zerosD = jnp.zeros((ROWS_SUB, D), jnp.float32)
    b1r = b1.reshape(1, D)
    b2r = b2.reshape(1, D)

    h1 = pl.pallas_call(
        _tc_matmul,
        out_shape=jax.ShapeDtypeStruct((N, D), jnp.float32),
    )(x, W1)
    cnt = _sc_degree(edges, onesD, zerosD)
    g1 = _call_tc(_tc_scale, cnt, h1)
    a1 = _sc_aggregate(g1, edges, zerosD)
    g2 = _call_tc(_tc_mid, cnt, a1, g1, b1r, W2)
    a2 = _sc_aggregate(g2, edges, zerosD)
    out = _call_tc(_tc_last, cnt, a2, g2, b2r)
    return out[:N]


# speedup vs baseline: 1.0261x; 1.0119x over previous
import functools

import jax
import jax.numpy as jnp
from jax import lax
from jax.experimental import pallas as pl
from jax.experimental.pallas import tpu as pltpu
from jax.experimental.pallas import tpu_sc as plsc

N = 10000
E = 320000
D = 128

NC = 2
NS = 16
NW = NC * NS
CHUNK = 125
CHUNKS = 80
E_TILE = CHUNK * CHUNKS
ROWS_SUB = 640
N_PAD = NS * ROWS_SUB
NBUF = 2
LOOK = 1
H = 2
CH = CHUNKS // H
DEG_RING = 4
assert CH % NBUF == 0 and 1 <= LOOK < NBUF

_MESH = plsc.VectorSubcoreMesh(core_axis_name="core", subcore_axis_name="subcore")


@functools.partial(
    pl.kernel,
    out_type=jax.ShapeDtypeStruct((NC, N_PAD, D), jnp.float32),
    mesh=_MESH,
    scratch_types=[
        pltpu.VMEM((CHUNKS, CHUNK), jnp.int32),
        pltpu.VMEM((CHUNK, D), jnp.float32),
        pltpu.VMEM_SHARED((N_PAD, D), jnp.float32),
    ] + [pltpu.SemaphoreType.DMA] * DEG_RING,
)
def _sc_degree(edges_hbm, ones_hbm, zeros_hbm, out_hbm, dst_v, ones_v, acc,
               *sems):
    c = lax.axis_index("core")
    s = lax.axis_index("subcore")
    wid = c * NS + s
    pltpu.sync_copy(zeros_hbm, acc.at[pl.ds(s * ROWS_SUB, ROWS_SUB)])
    pltpu.sync_copy(ones_hbm, ones_v)
    pltpu.sync_copy(edges_hbm.at[1, wid], dst_v)
    plsc.subcore_barrier()

    @pl.loop(0, CHUNKS, step=DEG_RING)
    def _(jo):
        for b in range(DEG_RING):
            j = jo + b

            @pl.when(jo > 0)
            def _():
                pltpu.make_async_copy(ones_v, acc.at[dst_v.at[j]],
                                      sems[b]).wait()

            pltpu.async_copy(ones_v, acc.at[dst_v.at[j]], sems[b], add=True)

    for b in range(DEG_RING):
        pltpu.make_async_copy(ones_v, acc.at[dst_v.at[b]], sems[b]).wait()

    plsc.subcore_barrier()
    pltpu.sync_copy(acc.at[pl.ds(s * ROWS_SUB, ROWS_SUB)],
                    out_hbm.at[c, pl.ds(s * ROWS_SUB, ROWS_SUB)])


@functools.partial(
    pl.kernel,
    out_type=jax.ShapeDtypeStruct((NC, N_PAD, D), jnp.float32),
    mesh=_MESH,
    scratch_types=[
        pltpu.VMEM((CH, CHUNK), jnp.int32),
        pltpu.VMEM((CH, CHUNK), jnp.int32),
        pltpu.VMEM((NBUF, CHUNK, D), jnp.float32),
        pltpu.VMEM_SHARED((N_PAD, D), jnp.float32),
    ] + [pltpu.SemaphoreType.DMA] * (2 * NBUF),
)
def _sc_aggregate(g_hbm, edges_hbm, zeros_hbm, out_hbm,
                  src_v, dst_v, rows_v, acc, *sems):
    gsem = sems[:NBUF]
    ssem = sems[NBUF:]
    c = lax.axis_index("core")
    s = lax.axis_index("subcore")
    wid = c * NS + s
    pltpu.sync_copy(zeros_hbm, acc.at[pl.ds(s * ROWS_SUB, ROWS_SUB)])
    plsc.subcore_barrier()

    for h in range(H):
        pltpu.sync_copy(edges_hbm.at[0, wid, pl.ds(h * CH, CH)], src_v)
        pltpu.sync_copy(edges_hbm.at[1, wid, pl.ds(h * CH, CH)], dst_v)
        for b in range(LOOK):
            pltpu.async_copy(g_hbm.at[src_v.at[b]], rows_v.at[b], gsem[b])

        @pl.loop(0, CH, step=NBUF)
        def _(jo):
            for b in range(NBUF):
                j = jo + b
                bl = (b + LOOK) % NBUF

                @pl.when(j >= NBUF - LOOK)
                def _():
                    pltpu.make_async_copy(rows_v.at[bl], acc.at[dst_v.at[j]],
                                          ssem[bl]).wait()

                jg = jnp.where(j + LOOK >= CH, 0, j + LOOK)
                pltpu.async_copy(g_hbm.at[src_v.at[jg]], rows_v.at[bl],
                                 gsem[bl])
                pltpu.make_async_copy(g_hbm.at[src_v.at[j]], rows_v.at[b],
                                      gsem[b]).wait()
                pltpu.async_copy(rows_v.at[b], acc.at[dst_v.at[j]], ssem[b],
                                 add=True)

        for t in range(CH + LOOK - NBUF, CH):
            pltpu.make_async_copy(rows_v.at[t % NBUF], acc.at[dst_v.at[0]],
                                  ssem[t % NBUF]).wait()
        for t in range(CH, CH + LOOK):
            pltpu.make_async_copy(g_hbm.at[src_v.at[0]], rows_v.at[t % NBUF],
                                  gsem[t % NBUF]).wait()

    plsc.subcore_barrier()
    pltpu.sync_copy(acc.at[pl.ds(s * ROWS_SUB, ROWS_SUB)],
                    out_hbm.at[c, pl.ds(s * ROWS_SUB, ROWS_SUB)])


def _dinv_from_counts(cnt_ref):
    s = (cnt_ref[0] + cnt_ref[1])[:, 0:1].astype(jnp.float32)
    return lax.rsqrt(1.0 + s)


def _tc_first(cnt_ref, x_ref, w_ref, g_ref):
    dinv = _dinv_from_counts(cnt_ref)
    h = jnp.dot(x_ref[...], w_ref[...], preferred_element_type=jnp.float32)
    g_ref[pl.ds(0, N)] = dinv[:N] * h
    g_ref[pl.ds(N, N_PAD - N)] = jnp.zeros((N_PAD - N, D), jnp.float32)


def _tc_mid(cnt_ref, a_ref, g_ref, b_ref, w_ref, g2_ref):
    dinv = _dinv_from_counts(cnt_ref)
    z = dinv * (a_ref[0] + a_ref[1] + g_ref[...]) + b_ref[...]
    z = jnp.maximum(z, 0.0)
    h = jnp.dot(z, w_ref[...], preferred_element_type=jnp.float32)
    g2_ref[...] = dinv * h


def _tc_last(cnt_ref, a_ref, g_ref, b_ref, out_ref):
    dinv = _dinv_from_counts(cnt_ref)
    out_ref[...] = dinv * (a_ref[0] + a_ref[1] + g_ref[...]) + b_ref[...]


def _call_tc(body, *args):
    return pl.pallas_call(
        body,
        out_shape=jax.ShapeDtypeStruct((N_PAD, D), jnp.float32),
    )(*args)


def kernel(x, edge_index, W1, b1, W2, b2):
    edges = edge_index.astype(jnp.int32).reshape(2, NW, CHUNKS, CHUNK)

    onesD = jnp.ones((CHUNK, D), jnp.float32)
    zerosD = jnp.zeros((ROWS_SUB, D), jnp.float32)
    b1r = b1.reshape(1, D)
    b2r = b2.reshape(1, D)

    cnt = _sc_degree(edges, onesD, zerosD)
    g1 = _call_tc(_tc_first, cnt, x, W1)
    a1 = _sc_aggregate(g1, edges, zerosD)
    g2 = _call_tc(_tc_mid, cnt, a1, g1, b1r, W2)
    a2 = _sc_aggregate(g2, edges, zerosD)
    out = _call_tc(_tc_last, cnt, a2, g2, b2r)
    return out[:N]
